# baseline (speedup 1.0000x reference)
import functools

import jax
import jax.numpy as jnp
from jax import lax
from jax.experimental import pallas as pl
from jax.experimental.pallas import tpu as pltpu
from jax.experimental.pallas import tpu_sc as plsc

_NC = 2
_NS = 16
_NW = _NC * _NS
_LANES = 16
_EMBED = 64
_CHUNK = 128


def _make_kernel(batch):
    bpw = batch // _NW
    n_chunks = bpw // _CHUNK
    mesh = plsc.VectorSubcoreMesh(
        core_axis_name="c", subcore_axis_name="s",
        num_cores=_NC, num_subcores=_NS)

    @functools.partial(
        pl.kernel,
        mesh=mesh,
        compiler_params=pltpu.CompilerParams(
            needs_layout_passes=False, use_tc_tiling_on_sc=False),
        out_type=jax.ShapeDtypeStruct((batch,), jnp.float32),
        scratch_types=[
            pltpu.VMEM((n_chunks, _CHUNK), jnp.int32),
            pltpu.VMEM((n_chunks, _CHUNK), jnp.int32),
            pltpu.VMEM((n_chunks, _CHUNK), jnp.int32),
            pltpu.VMEM((bpw, _EMBED), jnp.float32),
            pltpu.VMEM((bpw, _EMBED), jnp.float32),
            pltpu.VMEM((bpw, _EMBED), jnp.float32),
            pltpu.VMEM((bpw,), jnp.float32),
            pltpu.VMEM((_LANES, _LANES), jnp.float32),
            pltpu.SemaphoreType.DMA,
        ],
    )
    def trans_score(head_hbm, rel_hbm, tail_hbm, ent_hbm, relw_hbm, out_hbm,
                    hidx, ridx, tidx, hrows, rrows, trows, outv, tscr, sem):
        wid = lax.axis_index("s") * _NC + lax.axis_index("c")
        pltpu.sync_copy(head_hbm.at[wid], hidx)
        pltpu.sync_copy(rel_hbm.at[wid], ridx)
        pltpu.sync_copy(tail_hbm.at[wid], tidx)

        copies = []
        for c in range(n_chunks):
            rows = pl.ds(c * _CHUNK, _CHUNK)
            copies.append(pltpu.async_copy(ent_hbm.at[hidx.at[c]], hrows.at[rows], sem))
            copies.append(pltpu.async_copy(ent_hbm.at[tidx.at[c]], trows.at[rows], sem))
            copies.append(pltpu.async_copy(relw_hbm.at[ridx.at[c]], rrows.at[rows], sem))
        for cp in copies:
            cp.wait()

        lanes = lax.iota(jnp.int32, _LANES)

        def group_body(g, _):
            sv = jnp.zeros((_LANES,), jnp.float32)
            for r16 in range(_LANES):
                r = g * _LANES + r16
                acc = jnp.zeros((_LANES,), jnp.float32)
                for j in range(_EMBED // _LANES):
                    sl = pl.ds(j * _LANES, _LANES)
                    acc = acc + jnp.abs(hrows[r, sl] + rrows[r, sl] - trows[r, sl])
                sv = jnp.where(lanes == r16, jnp.sum(acc), sv)
            outv[pl.ds(g * _LANES, _LANES)] = sv
            return 0

        lax.fori_loop(0, bpw // _LANES, group_body, 0)
        pltpu.sync_copy(outv, out_hbm.at[pl.ds(wid * bpw, bpw)])

    return trans_score


def kernel(head, relation, tail, entity_weight, relation_weight):
    batch = head.shape[0]
    bpw = batch // _NW
    n_chunks = bpw // _CHUNK
    shape3 = (_NW, n_chunks, _CHUNK)
    fn = _make_kernel(batch)
    return fn(head.reshape(shape3), relation.reshape(shape3),
              tail.reshape(shape3), entity_weight, relation_weight)


# speedup vs baseline: 2.4232x; 2.4232x over previous
import functools

import jax
import jax.numpy as jnp
from jax import lax
from jax.experimental import pallas as pl
from jax.experimental.pallas import tpu as pltpu
from jax.experimental.pallas import tpu_sc as plsc

_NC = 2
_NS = 16
_NW = _NC * _NS
_LANES = 16
_EMBED = 64
_TILE = 8
_CHUNK = 32


def _make_kernel(batch):
    bpw = batch // _NW
    n_chunks = bpw // _CHUNK
    mesh = plsc.VectorSubcoreMesh(
        core_axis_name="c", subcore_axis_name="s",
        num_cores=_NC, num_subcores=_NS)

    @functools.partial(
        pl.kernel,
        mesh=mesh,
        compiler_params=pltpu.CompilerParams(
            needs_layout_passes=False, use_tc_tiling_on_sc=True),
        out_type=jax.ShapeDtypeStruct((batch,), jnp.float32),
        scratch_types=[
            pltpu.VMEM((bpw,), jnp.int32),
            pltpu.VMEM((bpw,), jnp.int32),
            pltpu.VMEM((bpw,), jnp.int32),
            pltpu.VMEM((_CHUNK, _EMBED), jnp.float32),
            pltpu.VMEM((_CHUNK, _EMBED), jnp.float32),
            pltpu.VMEM((_CHUNK, _EMBED), jnp.float32),
            pltpu.VMEM((bpw,), jnp.float32),
            pltpu.SemaphoreType.DMA,
        ],
    )
    def trans_score(head_hbm, rel_hbm, tail_hbm, ent_hbm, relw_hbm, out_hbm,
                    hidx, ridx, tidx, hbuf, rbuf, tbuf, outv, sem):
        wid = lax.axis_index("s") * _NC + lax.axis_index("c")
        pltpu.sync_copy(head_hbm.at[wid], hidx)
        pltpu.sync_copy(rel_hbm.at[wid], ridx)
        pltpu.sync_copy(tail_hbm.at[wid], tidx)

        lanes = lax.iota(jnp.int32, _LANES)

        def chunk_body(c, _):
            cps = []
            for g in range(_CHUNK // _LANES):
                base = c * _CHUNK + g * _LANES
                hv = hidx[pl.ds(base, _LANES)]
                rv = ridx[pl.ds(base, _LANES)]
                tv = tidx[pl.ds(base, _LANES)]
                for l in range(_LANES):
                    i = g * _LANES + l
                    h = hv[l]
                    cps.append(pltpu.async_copy(
                        ent_hbm.at[h >> 3, h & (_TILE - 1)], hbuf.at[i], sem))
                    r = rv[l]
                    cps.append(pltpu.async_copy(
                        relw_hbm.at[r >> 3, r & (_TILE - 1)], rbuf.at[i], sem))
                    t = tv[l]
                    cps.append(pltpu.async_copy(
                        ent_hbm.at[t >> 3, t & (_TILE - 1)], tbuf.at[i], sem))
            for cp in cps:
                cp.wait()

            for g in range(_CHUNK // _LANES):
                sv = jnp.zeros((_LANES,), jnp.float32)
                for r16 in range(_LANES):
                    i = g * _LANES + r16
                    acc = jnp.zeros((_LANES,), jnp.float32)
                    for j in range(_EMBED // _LANES):
                        sl = pl.ds(j * _LANES, _LANES)
                        acc = acc + jnp.abs(hbuf[i, sl] + rbuf[i, sl]
                                            - tbuf[i, sl])
                    sv = jnp.where(lanes == r16, jnp.sum(acc), sv)
                outv[pl.ds(c * _CHUNK + g * _LANES, _LANES)] = sv
            return 0

        lax.fori_loop(0, n_chunks, chunk_body, 0)
        pltpu.sync_copy(outv, out_hbm.at[pl.ds(wid * bpw, bpw)])

    return trans_score


def kernel(head, relation, tail, entity_weight, relation_weight):
    batch = head.shape[0]
    bpw = batch // _NW
    n_chunks = bpw // _CHUNK
    shape2 = (_NW, bpw)
    n_ent, emb = entity_weight.shape
    n_rel = relation_weight.shape[0]
    ent3d = entity_weight.reshape(n_ent // _TILE, _TILE, emb)
    rel3d = relation_weight.reshape(n_rel // _TILE, _TILE, emb)
    fn = _make_kernel(batch)
    return fn(head.reshape(shape2), relation.reshape(shape2),
              tail.reshape(shape2), ent3d, rel3d)
